# baseline (speedup 1.0000x reference)
import jax
import jax.numpy as jnp
from jax import lax
from jax.experimental import pallas as pl
from jax.experimental.pallas import tpu as pltpu

B = 4
C = 384
H = 24
W = 24
HW = H * W
N = B * HW
E = 8
F = 768
GROUPS = 32
CPG = C // GROUPS
EPS = 1e-6
EPAIR = 2
NEP = E // EPAIR
NPROG = B + NEP + 1


def _group_stats(x, gmat):
    s = jnp.sum(x, axis=0, keepdims=True)
    sq = jnp.sum(x * x, axis=0, keepdims=True)
    denom = float(CPG * HW)
    mean = jnp.dot(s, gmat, preferred_element_type=jnp.float32) / denom
    ex2 = jnp.dot(sq, gmat, preferred_element_type=jnp.float32) / denom
    return mean, ex2 - mean * mean


def _gn_silu(x, gmat, scale, bias):
    mean, var = _group_stats(x, gmat)
    xh = (x - mean) * lax.rsqrt(var + EPS) * scale + bias
    return xh * lax.logistic(xh)


def _conv3x3(a_bf16, w_ref):
    a3 = jnp.pad(a_bf16.reshape(H, W, C), ((1, 1), (1, 1), (0, 0)))
    acc = jnp.zeros((HW, C), jnp.float32)
    for k in range(9):
        dy, dx = k // 3, k % 3
        win = a3[dy:dy + H, dx:dx + W].reshape(HW, C)
        acc = acc + jnp.dot(win, w_ref[k], preferred_element_type=jnp.float32)
    return acc


def _gelu_tanh(g):
    c = 0.7978845608028654
    return 0.5 * g * (1.0 + jnp.tanh(c * (g + 0.044715 * g * g * g)))


def _mega_body(x_ref, w1_ref, w2_ref, gn1s_ref, gn1b_ref, c1b_ref,
               gn2s_ref, gn2b_ref, c2b_ref, rw_ref,
               egw_ref, euw_ref, edw_ref, egb_ref, eub_ref, edb_ref,
               sgw_ref, sgb_ref, suw_ref, sub_ref, sdw_ref, sdb_ref,
               out_ref, t_s, r_s, comb_s):
    f32 = jnp.float32
    bf16 = jnp.bfloat16
    i = pl.program_id(0)

    @pl.when(i < B)
    def _resnet():
        x = x_ref[0]
        ii = lax.broadcasted_iota(jnp.int32, (C, C), 0) // CPG
        jj = lax.broadcasted_iota(jnp.int32, (C, C), 1) // CPG
        gmat = (ii == jj).astype(f32)

        a1 = _gn_silu(x, gmat, gn1s_ref[...], gn1b_ref[...]).astype(bf16)
        h1 = _conv3x3(a1, w1_ref) + c1b_ref[...]
        a2 = _gn_silu(h1, gmat, gn2s_ref[...], gn2b_ref[...]).astype(bf16)
        h2 = _conv3x3(a2, w2_ref) + c2b_ref[...]
        r = x + h2

        logits = jnp.dot(r, rw_ref[...], preferred_element_type=f32)
        m = jnp.max(logits, axis=1, keepdims=True)
        ex = jnp.exp(logits - m)
        probs = ex / jnp.sum(ex, axis=1, keepdims=True)
        lane = lax.broadcasted_iota(jnp.int32, (HW, E), 1)
        v1 = jnp.max(probs, axis=1, keepdims=True)
        i1 = jnp.min(jnp.where(probs == v1, lane, E), axis=1, keepdims=True)
        p2 = jnp.where(lane == i1, -jnp.inf, probs)
        v2 = jnp.max(p2, axis=1, keepdims=True)
        i2 = jnp.min(jnp.where(p2 == v2, lane, E), axis=1, keepdims=True)
        s = v1 + v2
        comb = (jnp.where(lane == i1, v1 / s, 0.0)
                + jnp.where(lane == i2, v2 / s, 0.0))

        base = i * HW
        r_s[pl.ds(base, HW), :] = r
        t_s[pl.ds(base, HW), :] = r.astype(bf16)
        comb_s[pl.ds(base, HW), :] = comb

    @pl.when((i >= B) & (i < B + NEP))
    def _expert():
        ep = i - B
        t = t_s[...]
        comb = comb_s[...]
        lane = lax.broadcasted_iota(jnp.int32, (N, E), 1)
        acc = jnp.zeros((N, C), f32)
        for k in range(EPAIR):
            g = jnp.dot(t, egw_ref[k].astype(bf16), preferred_element_type=f32) + egb_ref[k]
            u = jnp.dot(t, euw_ref[k].astype(bf16), preferred_element_type=f32) + eub_ref[k]
            hh = (_gelu_tanh(g) * u).astype(bf16)
            o = jnp.dot(hh, edw_ref[k].astype(bf16), preferred_element_type=f32) + edb_ref[k]
            c = jnp.sum(jnp.where(lane == ep * EPAIR + k, comb, 0.0),
                        axis=1, keepdims=True)
            acc = acc + o * c

        @pl.when(ep == 0)
        def _init():
            out_ref[...] = r_s[...] + acc

        @pl.when(ep != 0)
        def _acc():
            out_ref[...] = out_ref[...] + acc

    @pl.when(i == B + NEP)
    def _shared():
        t = t_s[...]
        g = jnp.dot(t, sgw_ref[...].astype(bf16), preferred_element_type=f32) + sgb_ref[...]
        u = jnp.dot(t, suw_ref[...].astype(bf16), preferred_element_type=f32) + sub_ref[...]
        hh = (_gelu_tanh(g) * u).astype(bf16)
        o = jnp.dot(hh, sdw_ref[...].astype(bf16), preferred_element_type=f32) + sdb_ref[...]
        out_ref[...] = out_ref[...] + o


@jax.jit
def kernel(x, gn1_s, gn1_b, conv1_w, conv1_b, gn2_s, gn2_b, conv2_w, conv2_b,
           router_w, eg_w, eg_b, eu_w, eu_b, ed_w, ed_b,
           sg_w, sg_b, su_w, su_b, sd_w, sd_b):
    f32 = jnp.float32
    bf16 = jnp.bfloat16
    xt = x.transpose(0, 2, 3, 1).reshape(B, HW, C)
    w1m = conv1_w.transpose(2, 3, 1, 0).reshape(9, C, C).astype(bf16)
    w2m = conv2_w.transpose(2, 3, 1, 0).reshape(9, C, C).astype(bf16)

    cvec = lambda: pl.BlockSpec((1, C), lambda i: (0, 0))
    fvec = lambda: pl.BlockSpec((1, F), lambda i: (0, 0))
    eidx = lambda i: (jnp.clip(i - B, 0, NEP - 1), 0, 0)

    mega = pl.pallas_call(
        _mega_body,
        grid=(NPROG,),
        in_specs=[
            pl.BlockSpec((1, HW, C), lambda i: (jnp.minimum(i, B - 1), 0, 0)),
            pl.BlockSpec((9, C, C), lambda i: (0, 0, 0)),
            pl.BlockSpec((9, C, C), lambda i: (0, 0, 0)),
            cvec(), cvec(), cvec(), cvec(), cvec(), cvec(),
            pl.BlockSpec((C, E), lambda i: (0, 0)),
            pl.BlockSpec((EPAIR, C, F), eidx),
            pl.BlockSpec((EPAIR, C, F), eidx),
            pl.BlockSpec((EPAIR, F, C), eidx),
            pl.BlockSpec((EPAIR, 1, F), eidx),
            pl.BlockSpec((EPAIR, 1, F), eidx),
            pl.BlockSpec((EPAIR, 1, C), eidx),
            pl.BlockSpec((C, F), lambda i: (0, 0)),
            fvec(),
            pl.BlockSpec((C, F), lambda i: (0, 0)),
            fvec(),
            pl.BlockSpec((F, C), lambda i: (0, 0)),
            cvec(),
        ],
        out_specs=pl.BlockSpec((N, C), lambda i: (0, 0)),
        out_shape=jax.ShapeDtypeStruct((N, C), f32),
        scratch_shapes=[
            pltpu.VMEM((N, C), bf16),
            pltpu.VMEM((N, C), f32),
            pltpu.VMEM((N, E), f32),
        ],
    )
    out = mega(
        xt, w1m, w2m,
        gn1_s.reshape(1, C), gn1_b.reshape(1, C), conv1_b.reshape(1, C),
        gn2_s.reshape(1, C), gn2_b.reshape(1, C), conv2_b.reshape(1, C),
        router_w,
        eg_w, eu_w, ed_w,
        eg_b.reshape(E, 1, F), eu_b.reshape(E, 1, F), ed_b.reshape(E, 1, C),
        sg_w, sg_b.reshape(1, F),
        su_w, su_b.reshape(1, F),
        sd_w, sd_b.reshape(1, C),
    )
    return out.reshape(B, H, W, C).transpose(0, 3, 1, 2)


# speedup vs baseline: 1.0085x; 1.0085x over previous
import jax
import jax.numpy as jnp
from jax import lax
from jax.experimental import pallas as pl
from jax.experimental.pallas import tpu as pltpu

B = 4
C = 384
H = 24
W = 24
HW = H * W
N = B * HW
E = 8
F = 768
GROUPS = 32
CPG = C // GROUPS
EPS = 1e-6
NPROG = B + E + 1


def _group_stats(x, gmat):
    s = jnp.sum(x, axis=0, keepdims=True)
    sq = jnp.sum(x * x, axis=0, keepdims=True)
    denom = float(CPG * HW)
    mean = jnp.dot(s, gmat, preferred_element_type=jnp.float32) / denom
    ex2 = jnp.dot(sq, gmat, preferred_element_type=jnp.float32) / denom
    return mean, ex2 - mean * mean


def _gn_silu(x, gmat, scale, bias):
    mean, var = _group_stats(x, gmat)
    xh = (x - mean) * lax.rsqrt(var + EPS) * scale + bias
    return xh * lax.logistic(xh)


def _conv3x3(a_bf16, w_ref):
    a3 = jnp.pad(a_bf16.reshape(H, W, C), ((1, 1), (1, 1), (0, 0)))
    acc = jnp.zeros((HW, C), jnp.float32)
    for k in range(9):
        dy, dx = k // 3, k % 3
        win = a3[dy:dy + H, dx:dx + W].reshape(HW, C)
        acc = acc + jnp.dot(win, w_ref[k], preferred_element_type=jnp.float32)
    return acc


def _gelu_tanh(g):
    c = 0.7978845608028654
    return 0.5 * g * (1.0 + jnp.tanh(c * (g + 0.044715 * g * g * g)))


def _mega_body(x_ref, w1_ref, w2_ref, gn1s_ref, gn1b_ref, c1b_ref,
               gn2s_ref, gn2b_ref, c2b_ref, rw_ref,
               egw_ref, euw_ref, edw_ref, egb_ref, eub_ref, edb_ref,
               sgw_ref, sgb_ref, suw_ref, sub_ref, sdw_ref, sdb_ref,
               out_ref, t_s, r_s, comb_s):
    f32 = jnp.float32
    bf16 = jnp.bfloat16
    i = pl.program_id(0)

    @pl.when(i < B)
    def _resnet():
        x = x_ref[0]
        ii = lax.broadcasted_iota(jnp.int32, (C, C), 0) // CPG
        jj = lax.broadcasted_iota(jnp.int32, (C, C), 1) // CPG
        gmat = (ii == jj).astype(f32)

        a1 = _gn_silu(x, gmat, gn1s_ref[...], gn1b_ref[...]).astype(bf16)
        h1 = _conv3x3(a1, w1_ref) + c1b_ref[...]
        a2 = _gn_silu(h1, gmat, gn2s_ref[...], gn2b_ref[...]).astype(bf16)
        h2 = _conv3x3(a2, w2_ref) + c2b_ref[...]
        r = x + h2

        logits = jnp.dot(r, rw_ref[...], preferred_element_type=f32)
        m = jnp.max(logits, axis=1, keepdims=True)
        ex = jnp.exp(logits - m)
        probs = ex / jnp.sum(ex, axis=1, keepdims=True)
        lane = lax.broadcasted_iota(jnp.int32, (HW, E), 1)
        v1 = jnp.max(probs, axis=1, keepdims=True)
        i1 = jnp.min(jnp.where(probs == v1, lane, E), axis=1, keepdims=True)
        p2 = jnp.where(lane == i1, -jnp.inf, probs)
        v2 = jnp.max(p2, axis=1, keepdims=True)
        i2 = jnp.min(jnp.where(p2 == v2, lane, E), axis=1, keepdims=True)
        s = v1 + v2
        comb = (jnp.where(lane == i1, v1 / s, 0.0)
                + jnp.where(lane == i2, v2 / s, 0.0))

        base = i * HW
        r_s[pl.ds(base, HW), :] = r
        t_s[pl.ds(base, HW), :] = r.astype(bf16)
        comb_s[pl.ds(base, HW), :] = comb

    @pl.when((i >= B) & (i < B + E))
    def _expert():
        e = i - B
        t = t_s[...]
        g = jnp.dot(t, egw_ref[0].astype(bf16), preferred_element_type=f32) + egb_ref[0]
        u = jnp.dot(t, euw_ref[0].astype(bf16), preferred_element_type=f32) + eub_ref[0]
        hh = (_gelu_tanh(g) * u).astype(bf16)
        o = jnp.dot(hh, edw_ref[0].astype(bf16), preferred_element_type=f32) + edb_ref[0]
        lane = lax.broadcasted_iota(jnp.int32, (N, E), 1)
        c = jnp.sum(jnp.where(lane == e, comb_s[...], 0.0),
                    axis=1, keepdims=True)
        contrib = o * c

        @pl.when(e == 0)
        def _init():
            out_ref[...] = r_s[...] + contrib

        @pl.when(e != 0)
        def _acc():
            out_ref[...] = out_ref[...] + contrib

    @pl.when(i == B + E)
    def _shared():
        t = t_s[...]
        g = jnp.dot(t, sgw_ref[...].astype(bf16), preferred_element_type=f32) + sgb_ref[...]
        u = jnp.dot(t, suw_ref[...].astype(bf16), preferred_element_type=f32) + sub_ref[...]
        hh = (_gelu_tanh(g) * u).astype(bf16)
        o = jnp.dot(hh, sdw_ref[...].astype(bf16), preferred_element_type=f32) + sdb_ref[...]
        out_ref[...] = out_ref[...] + o


@jax.jit
def kernel(x, gn1_s, gn1_b, conv1_w, conv1_b, gn2_s, gn2_b, conv2_w, conv2_b,
           router_w, eg_w, eg_b, eu_w, eu_b, ed_w, ed_b,
           sg_w, sg_b, su_w, su_b, sd_w, sd_b):
    f32 = jnp.float32
    bf16 = jnp.bfloat16
    xt = x.transpose(0, 2, 3, 1).reshape(B, HW, C)
    w1m = conv1_w.transpose(2, 3, 1, 0).reshape(9, C, C).astype(bf16)
    w2m = conv2_w.transpose(2, 3, 1, 0).reshape(9, C, C).astype(bf16)

    cvec = lambda: pl.BlockSpec((1, C), lambda i: (0, 0))
    fvec = lambda: pl.BlockSpec((1, F), lambda i: (0, 0))
    eidx = lambda i: (jnp.clip(i - B, 0, E - 1), 0, 0)

    mega = pl.pallas_call(
        _mega_body,
        grid=(NPROG,),
        in_specs=[
            pl.BlockSpec((1, HW, C), lambda i: (jnp.minimum(i, B - 1), 0, 0)),
            pl.BlockSpec((9, C, C), lambda i: (0, 0, 0)),
            pl.BlockSpec((9, C, C), lambda i: (0, 0, 0)),
            cvec(), cvec(), cvec(), cvec(), cvec(), cvec(),
            pl.BlockSpec((C, E), lambda i: (0, 0)),
            pl.BlockSpec((1, C, F), eidx),
            pl.BlockSpec((1, C, F), eidx),
            pl.BlockSpec((1, F, C), eidx),
            pl.BlockSpec((1, 1, F), eidx),
            pl.BlockSpec((1, 1, F), eidx),
            pl.BlockSpec((1, 1, C), eidx),
            pl.BlockSpec((C, F), lambda i: (0, 0)),
            fvec(),
            pl.BlockSpec((C, F), lambda i: (0, 0)),
            fvec(),
            pl.BlockSpec((F, C), lambda i: (0, 0)),
            cvec(),
        ],
        out_specs=pl.BlockSpec((N, C), lambda i: (0, 0)),
        out_shape=jax.ShapeDtypeStruct((N, C), f32),
        scratch_shapes=[
            pltpu.VMEM((N, C), bf16),
            pltpu.VMEM((N, C), f32),
            pltpu.VMEM((N, E), f32),
        ],
    )
    out = mega(
        xt, w1m, w2m,
        gn1_s.reshape(1, C), gn1_b.reshape(1, C), conv1_b.reshape(1, C),
        gn2_s.reshape(1, C), gn2_b.reshape(1, C), conv2_b.reshape(1, C),
        router_w,
        eg_w, eu_w, ed_w,
        eg_b.reshape(E, 1, F), eu_b.reshape(E, 1, F), ed_b.reshape(E, 1, C),
        sg_w, sg_b.reshape(1, F),
        su_w, su_b.reshape(1, F),
        sd_w, sd_b.reshape(1, C),
    )
    return out.reshape(B, H, W, C).transpose(0, 3, 1, 2)
